# baseline (speedup 1.0000x reference)
import functools

import jax
import jax.numpy as jnp
from jax import lax
from jax.experimental import pallas as pl
from jax.experimental.pallas import tpu as pltpu
from jax.experimental.pallas import tpu_sc as plsc

_LANES = 16
_D = 64
_B = 16384
_IDX_W = 128
_NW = 32
_B_PER_W = _B // _NW
_CHUNKS = _B_PER_W // _IDX_W


def _tec_body(users_hbm, items_hbm, utable_hbm, itable_hbm, out_hbm,
              uidx_v, iidx_v, urows_v, irows_v, out_v, usem, isem):
    wid = lax.axis_index("s") * 2 + lax.axis_index("c")
    idx_row0 = wid * _CHUNKS

    pltpu.sync_copy(users_hbm.at[pl.ds(idx_row0, _CHUNKS)], uidx_v)
    pltpu.sync_copy(items_hbm.at[pl.ds(idx_row0, _CHUNKS)], iidx_v)

    ucopies = []
    icopies = []
    for j in range(_CHUNKS):
        rsl = pl.ds(j * _IDX_W, _IDX_W)
        ucopies.append(
            pltpu.async_copy(utable_hbm.at[uidx_v.at[j]], urows_v.at[rsl], usem))
        icopies.append(
            pltpu.async_copy(itable_hbm.at[iidx_v.at[j]], irows_v.at[rsl], isem))
    for c in ucopies + icopies:
        c.wait()

    lane = lax.iota(jnp.int32, _LANES)

    def group(g, _):
        r0 = g * _LANES
        sums = jnp.zeros((_LANES,), jnp.float32)
        for k in range(_LANES):
            acc = jnp.zeros((_LANES,), jnp.float32)
            for c in range(_D // _LANES):
                csl = pl.ds(c * _LANES, _LANES)
                acc = acc + urows_v[r0 + k, csl] * irows_v[r0 + k, csl]
            sums = jnp.where(lane == k, jnp.sum(acc), sums)
        out_v[pl.ds(r0, _LANES)] = 1.0 / (1.0 + jnp.exp(-sums))
        return _

    lax.fori_loop(0, _B_PER_W // _LANES, group, None)

    pltpu.sync_copy(out_v, out_hbm.at[pl.ds(wid * _B_PER_W, _B_PER_W)])


@jax.jit
def _pure_mf_sc(users2d, items2d, user_table, item_table):
    mesh = plsc.VectorSubcoreMesh(core_axis_name="c", subcore_axis_name="s")
    return pl.kernel(
        _tec_body,
        mesh=mesh,
        compiler_params=pltpu.CompilerParams(
            needs_layout_passes=False, use_tc_tiling_on_sc=False),
        out_type=jax.ShapeDtypeStruct((_B,), jnp.float32),
        scratch_types=[
            pltpu.VMEM((_CHUNKS, _IDX_W), jnp.int32),
            pltpu.VMEM((_CHUNKS, _IDX_W), jnp.int32),
            pltpu.VMEM((_B_PER_W, _D), jnp.float32),
            pltpu.VMEM((_B_PER_W, _D), jnp.float32),
            pltpu.VMEM((_B_PER_W,), jnp.float32),
            pltpu.SemaphoreType.DMA,
            pltpu.SemaphoreType.DMA,
        ],
    )(users2d, items2d, user_table, item_table)


def kernel(users, items, user_table, item_table):
    users2d = users.reshape(_B // _IDX_W, _IDX_W)
    items2d = items.reshape(_B // _IDX_W, _IDX_W)
    return _pure_mf_sc(users2d, items2d, user_table, item_table)


# speedup vs baseline: 1.5796x; 1.5796x over previous
import jax
import jax.numpy as jnp
from jax import lax
from jax.experimental import pallas as pl
from jax.experimental.pallas import tpu as pltpu
from jax.experimental.pallas import tpu_sc as plsc

_LANES = 16
_D = 64
_B = 16384
_NW = 32
_B_PER_W = _B // _NW
_HALF = _B_PER_W // 2


def _tec_body(users_hbm, items_hbm, utable_hbm, itable_hbm, out_hbm,
              uidx_v, iidx_v, urows_v, irows_v, out_v, usem, isem):
    wid = lax.axis_index("s") * 2 + lax.axis_index("c")
    base = wid * _B_PER_W

    pltpu.sync_copy(users_hbm.at[pl.ds(base, _B_PER_W)], uidx_v)
    pltpu.sync_copy(items_hbm.at[pl.ds(base, _B_PER_W)], iidx_v)

    lane = lax.iota(jnp.int32, _LANES)

    for half in range(2):
        h0 = half * _HALF

        def issue(g, _):
            r0 = g * _LANES
            uvec = uidx_v[pl.ds(h0 + r0, _LANES)]
            ivec = iidx_v[pl.ds(h0 + r0, _LANES)]
            for j in range(_LANES):
                pltpu.async_copy(utable_hbm.at[uvec[j]],
                                 urows_v.at[r0 + j], usem)
                pltpu.async_copy(itable_hbm.at[ivec[j]],
                                 irows_v.at[r0 + j], isem)
            return _

        lax.fori_loop(0, _HALF // _LANES, issue, None)

        def drain(r, _):
            pltpu.make_async_copy(utable_hbm.at[0], urows_v.at[0], usem).wait()
            pltpu.make_async_copy(itable_hbm.at[0], irows_v.at[0], isem).wait()
            return _

        lax.fori_loop(0, _HALF, drain, None)

        def group(g, _):
            r0 = g * _LANES
            sums = jnp.zeros((_LANES,), jnp.float32)
            for k in range(_LANES):
                acc = jnp.zeros((_LANES,), jnp.float32)
                for c in range(_D // _LANES):
                    csl = pl.ds(c * _LANES, _LANES)
                    acc = acc + urows_v[r0 + k, csl] * irows_v[r0 + k, csl]
                sums = jnp.where(lane == k, jnp.sum(acc), sums)
            out_v[pl.ds(h0 + r0, _LANES)] = 1.0 / (1.0 + jnp.exp(-sums))
            return _

        lax.fori_loop(0, _HALF // _LANES, group, None)

    pltpu.sync_copy(out_v, out_hbm.at[pl.ds(base, _B_PER_W)])


@jax.jit
def _pure_mf_sc(users, items, user_table, item_table):
    mesh = plsc.VectorSubcoreMesh(core_axis_name="c", subcore_axis_name="s")
    return pl.kernel(
        _tec_body,
        mesh=mesh,
        compiler_params=pltpu.CompilerParams(needs_layout_passes=False),
        out_type=jax.ShapeDtypeStruct((_B,), jnp.float32),
        scratch_types=[
            pltpu.VMEM((_B_PER_W,), jnp.int32),
            pltpu.VMEM((_B_PER_W,), jnp.int32),
            pltpu.VMEM((_HALF, _D), jnp.float32),
            pltpu.VMEM((_HALF, _D), jnp.float32),
            pltpu.VMEM((_B_PER_W,), jnp.float32),
            pltpu.SemaphoreType.DMA,
            pltpu.SemaphoreType.DMA,
        ],
    )(users, items, user_table, item_table)


def kernel(users, items, user_table, item_table):
    return _pure_mf_sc(users, items, user_table, item_table)
